# speedup vs baseline: 2.9738x; 2.9738x over previous
import jax
import jax.numpy as jnp
from jax import lax
from jax.experimental import pallas as pl
from jax.experimental.pallas import tpu as pltpu
from jax.experimental.pallas import tpu_sc as plsc

B_ROWS = 4096 * 50
D = 128
NC, NS = 2, 16
NW = NC * NS
B_PER_W = B_ROWS // NW
C = 128
NCHUNK = B_PER_W // C


def _emb_body(idx_hbm, table_hbm, out_hbm, idx_v, rows_v, gsem):
    wid = lax.axis_index("s") * NC + lax.axis_index("c")
    pltpu.sync_copy(idx_hbm.at[wid], idx_v)
    base = wid * B_PER_W

    def body(c, carry):
        row0 = base + c * C
        pltpu.async_copy(table_hbm.at[idx_v.at[c]], rows_v, gsem).wait()
        pltpu.sync_copy(rows_v, out_hbm.at[pl.ds(row0, C)])
        return carry

    lax.fori_loop(0, NCHUNK, body, 0)


def _run(idx_grp, table):
    f = pl.kernel(
        _emb_body,
        out_type=jax.ShapeDtypeStruct((B_ROWS, D), jnp.float32),
        mesh=plsc.VectorSubcoreMesh(core_axis_name="c", subcore_axis_name="s"),
        scratch_types=[
            pltpu.VMEM((NCHUNK, C), jnp.int32),
            pltpu.VMEM((C, D), jnp.float32),
            pltpu.SemaphoreType.DMA,
        ],
    )
    return f(idx_grp, table)


def kernel(idx, emb_weight):
    n, s = idx.shape
    idx_grp = idx.astype(jnp.int32).reshape(NW, NCHUNK, C)
    out = _run(idx_grp, emb_weight)
    return out.reshape(n, s, D)


# speedup vs baseline: 3.1224x; 1.0500x over previous
import jax
import jax.numpy as jnp
from jax import lax
from jax.experimental import pallas as pl
from jax.experimental.pallas import tpu as pltpu
from jax.experimental.pallas import tpu_sc as plsc

B_ROWS = 4096 * 50
D = 128
NC, NS = 2, 16
NW = NC * NS
B_PER_W = B_ROWS // NW
C = 128
NCHUNK = B_PER_W // C


def _emb_body(idx_hbm, table_hbm, out_hbm, idx_v, buf0, buf1, g0, g1, o0, o1):
    wid = lax.axis_index("s") * NC + lax.axis_index("c")
    pltpu.sync_copy(idx_hbm.at[wid], idx_v)
    base = wid * B_PER_W

    bufs = (buf0, buf1)
    gsems = (g0, g1)
    osems = (o0, o1)

    def gather(c, b):
        pltpu.async_copy(table_hbm.at[idx_v.at[c]], bufs[b], gsems[b])

    def wait_gather(c, b):
        pltpu.make_async_copy(table_hbm.at[idx_v.at[c]], bufs[b], gsems[b]).wait()

    def store(c, b):
        pltpu.async_copy(bufs[b], out_hbm.at[pl.ds(base + c * C, C)], osems[b])

    def wait_store(b):
        pltpu.make_async_copy(bufs[b], out_hbm.at[pl.ds(base, C)], osems[b]).wait()

    gather(0, 0)

    def body(g, carry):
        c0 = g * 2
        wait_gather(c0, 0)
        store(c0, 0)

        @pl.when(g >= 1)
        def _():
            wait_store(1)

        gather(c0 + 1, 1)

        wait_gather(c0 + 1, 1)
        store(c0 + 1, 1)
        wait_store(0)

        @pl.when(g < NCHUNK // 2 - 1)
        def _():
            gather(c0 + 2, 0)

        return carry

    lax.fori_loop(0, NCHUNK // 2, body, 0)
    wait_store(1)


def _run(idx_grp, table):
    f = pl.kernel(
        _emb_body,
        out_type=jax.ShapeDtypeStruct((B_ROWS, D), jnp.float32),
        mesh=plsc.VectorSubcoreMesh(core_axis_name="c", subcore_axis_name="s"),
        scratch_types=[
            pltpu.VMEM((NCHUNK, C), jnp.int32),
            pltpu.VMEM((C, D), jnp.float32),
            pltpu.VMEM((C, D), jnp.float32),
            pltpu.SemaphoreType.DMA,
            pltpu.SemaphoreType.DMA,
            pltpu.SemaphoreType.DMA,
            pltpu.SemaphoreType.DMA,
        ],
    )
    return f(idx_grp, table)


def kernel(idx, emb_weight):
    n, s = idx.shape
    idx_grp = idx.astype(jnp.int32).reshape(NW, NCHUNK, C)
    out = _run(idx_grp, emb_weight)
    return out.reshape(n, s, D)
